# baseline (speedup 1.0000x reference)
import functools

import jax
import jax.numpy as jnp
from jax.experimental import pallas as pl

N = 8192
TQ = 256
NRC = 512
K = 8


def _loss_kernel(pf_ref, pb_ref, gf_ref, gb_ref, out_ref):
    i = pl.program_id(0)

    pf = pf_ref[...]
    q2 = jnp.sum(pf * pf, axis=1, keepdims=True)

    def body(c, carry):
        mf, mv = carry
        c0 = c * NRC
        gf = gf_ref[:, pl.ds(c0, NRC)]
        gb = gb_ref[:, pl.ds(c0, NRC)]
        r2 = jnp.sum(gf * gf, axis=0, keepdims=True)
        cf = jax.lax.dot_general(
            pf, gf, (((1,), (0,)), ((), ())),
            preferred_element_type=jnp.float32,
            precision=jax.lax.Precision.HIGHEST)
        cb = jax.lax.dot_general(
            pb_ref[...], gb, (((1,), (0,)), ((), ())),
            preferred_element_type=jnp.float32)
        s = q2 + r2
        d2f = s - 2.0 * cb
        d2v = s - 2.0 * cf
        take = d2f < mf
        mf = jnp.where(take, d2f, mf)
        mv = jnp.where(take, d2v, mv)
        return mf, mv

    inf = jnp.full((TQ, NRC), jnp.inf, dtype=jnp.float32)
    mf, mv = jax.lax.fori_loop(0, N // NRC, body, (inf, inf))

    rowf = jnp.min(mf, axis=1, keepdims=True)
    rowv = jnp.min(jnp.where(mf == rowf, mv, jnp.inf), axis=1)
    part = jnp.sum(rowv).reshape(1, 1)

    @pl.when(i == 0)
    def _():
        out_ref[...] = jnp.zeros((1, 1), jnp.float32)

    out_ref[...] += part


@jax.jit
def _loss(pred, gt):
    p = jnp.pad(pred.reshape(N, 3), ((0, 0), (0, K - 3)))
    g = jnp.pad(gt.reshape(N, 3).T, ((0, K - 3), (0, 0)))

    total = pl.pallas_call(
        _loss_kernel,
        grid=(N // TQ,),
        in_specs=[
            pl.BlockSpec((TQ, K), lambda i: (i, 0)),
            pl.BlockSpec((TQ, K), lambda i: (i, 0)),
            pl.BlockSpec((K, N), lambda i: (0, 0)),
            pl.BlockSpec((K, N), lambda i: (0, 0)),
        ],
        out_specs=pl.BlockSpec((1, 1), lambda i: (0, 0)),
        out_shape=jax.ShapeDtypeStruct((1, 1), jnp.float32),
    )(p, p.astype(jnp.bfloat16), g, g.astype(jnp.bfloat16))
    return (total[0, 0] / (N * 3)).astype(jnp.float32)


def kernel(pred, gt):
    return _loss(pred, gt)


# speedup vs baseline: 1.1204x; 1.1204x over previous
import functools

import jax
import jax.numpy as jnp
from jax.experimental import pallas as pl

N = 8192
TQ = 256
NRC = 512
KB = 8
KC = 16
LW = 128


def _loss_kernel(lb_ref, lc_ref, rb_ref, rc_ref, p_ref, f_ref, c_ref, q2_ref):
    lb = lb_ref[...]
    lc = lc_ref[...]

    def body(c, carry):
        mf, mc = carry
        c0 = c * NRC
        kb = jax.lax.dot_general(
            lb, rb_ref[:, pl.ds(c0, NRC)], (((1,), (0,)), ((), ())),
            preferred_element_type=jnp.float32)
        kc = jax.lax.dot_general(
            lc, rc_ref[:, pl.ds(c0, NRC)], (((1,), (0,)), ((), ())),
            preferred_element_type=jnp.float32)
        for g in range(NRC // LW):
            s = slice(g * LW, (g + 1) * LW)
            kbg = kb[:, s]
            take = kbg < mf
            mf = jnp.where(take, kbg, mf)
            mc = jnp.where(take, kc[:, s], mc)
        return mf, mc

    inf = jnp.full((TQ, LW), jnp.inf, dtype=jnp.float32)
    zero = jnp.zeros((TQ, LW), dtype=jnp.float32)
    mf, mc = jax.lax.fori_loop(0, N // NRC, body, (inf, zero))

    rowf = jnp.min(mf, axis=1, keepdims=True)
    rowc = jnp.min(jnp.where(mf == rowf, mc, jnp.inf),
                   axis=1, keepdims=True)
    p = p_ref[...]
    q2 = jnp.sum(p * p, axis=1, keepdims=True)
    f_ref[...] = rowf
    c_ref[...] = rowc
    q2_ref[...] = q2


def _round_bf16_f32(x):
    u = jax.lax.bitcast_convert_type(x, jnp.uint32)
    r = (u + jnp.uint32(0x7FFF) + ((u >> 16) & jnp.uint32(1))) \
        & jnp.uint32(0xFFFF0000)
    return jax.lax.bitcast_convert_type(r, jnp.float32)


def _as_bf16(x):
    u = jax.lax.bitcast_convert_type(x, jnp.uint32)
    hi = (u >> 16).astype(jnp.uint16)
    return jax.lax.bitcast_convert_type(hi, jnp.bfloat16)


def _prep(pred, gt):
    q = pred.reshape(N, 3)
    g3 = gt.reshape(N, 3)
    bf16 = jnp.bfloat16

    qh = _round_bf16_f32(q)
    ql = _round_bf16_f32(q - qh)
    gh = _round_bf16_f32(g3)
    gl = _round_bf16_f32(g3 - gh)

    p = jnp.pad(q, ((0, 0), (0, KB - 3)))

    m2qh = _as_bf16(-2.0 * qh)
    m2ql = _as_bf16(-2.0 * ql)
    lb = jnp.zeros((N, KB), bf16)
    lb = lb.at[:, 0:3].set(m2qh)
    lb = lb.at[:, 3:6].set(1.0)

    lc = jnp.zeros((N, KC), bf16)
    lc = lc.at[:, 0:3].set(m2qh)
    lc = lc.at[:, 3:6].set(m2ql)
    lc = lc.at[:, 6:9].set(m2ql)

    r2 = jnp.sum(g3 * g3, axis=1)
    r2h = _round_bf16_f32(r2)
    r2l = _round_bf16_f32(r2 - r2h)
    r2l2 = _round_bf16_f32(r2 - r2h - r2l)
    rb = jnp.zeros((KB, N), bf16)
    rb = rb.at[0:3, :].set(_as_bf16(gh).T)
    rb = rb.at[3, :].set(_as_bf16(r2h))
    rb = rb.at[4, :].set(_as_bf16(r2l))
    rb = rb.at[5, :].set(_as_bf16(r2l2))

    glb = _as_bf16(gl)
    ghb = _as_bf16(gh)
    rc = jnp.zeros((KC, N), bf16)
    rc = rc.at[0:3, :].set(glb.T)
    rc = rc.at[3:6, :].set(ghb.T)
    rc = rc.at[6:9, :].set(glb.T)
    return lb, lc, rb, rc, p


@jax.jit
def _loss(pred, gt):
    lb, lc, rb, rc, p = _prep(pred, gt)
    tot = pl.pallas_call(
        _loss_kernel,
        grid=(N // TQ,),
        in_specs=[
            pl.BlockSpec((TQ, KB), lambda i: (i, 0)),
            pl.BlockSpec((TQ, KC), lambda i: (i, 0)),
            pl.BlockSpec((KB, N), lambda i: (0, 0)),
            pl.BlockSpec((KC, N), lambda i: (0, 0)),
            pl.BlockSpec((TQ, KB), lambda i: (i, 0)),
        ],
        out_specs=(pl.BlockSpec((TQ, 1), lambda i: (i, 0)),
                   pl.BlockSpec((TQ, 1), lambda i: (i, 0)),
                   pl.BlockSpec((TQ, 1), lambda i: (i, 0))),
        out_shape=(jax.ShapeDtypeStruct((N, 1), jnp.float32),
                   jax.ShapeDtypeStruct((N, 1), jnp.float32),
                   jax.ShapeDtypeStruct((N, 1), jnp.float32)),
    )(lb, lc, rb, rc, p)
    rowf, rowc, q2 = tot
    return (jnp.sum(rowf + rowc + q2) / (N * 3)).astype(jnp.float32)


def kernel(pred, gt):
    return _loss(pred, gt)


# speedup vs baseline: 1.4164x; 1.2642x over previous
import functools

import jax
import jax.numpy as jnp
from jax.experimental import pallas as pl

N = 8192
TQ = 256
NRC = 512
KB = 8
KC = 16
LW = 128


def _loss_kernel(lb_ref, lc_ref, rb_ref, rc_ref, p_ref, f_ref, c_ref, q2_ref):
    lb = lb_ref[...]
    lc = lc_ref[...]

    def combine(a, b):
        af, ac = a
        bf, bc = b
        take = bf < af
        return jnp.where(take, bf, af), jnp.where(take, bc, ac)

    mf = mc = None
    for c in range(N // NRC):
        c0 = c * NRC
        kb = jax.lax.dot_general(
            lb, rb_ref[:, pl.ds(c0, NRC)], (((1,), (0,)), ((), ())),
            preferred_element_type=jnp.float32)
        kc = jax.lax.dot_general(
            lc, rc_ref[:, pl.ds(c0, NRC)], (((1,), (0,)), ((), ())),
            preferred_element_type=jnp.float32)
        gs = [(kb[:, g * LW:(g + 1) * LW], kc[:, g * LW:(g + 1) * LW])
              for g in range(NRC // LW)]
        t01 = combine(gs[0], gs[1])
        t23 = combine(gs[2], gs[3])
        t = combine(t01, t23)
        if mf is None:
            mf, mc = t
        else:
            mf, mc = combine((mf, mc), t)

    rowf = jnp.min(mf, axis=1, keepdims=True)
    rowc = jnp.min(jnp.where(mf == rowf, mc, jnp.inf),
                   axis=1, keepdims=True)
    p = p_ref[...]
    q2 = jnp.sum(p * p, axis=1, keepdims=True)
    f_ref[...] = rowf
    c_ref[...] = rowc
    q2_ref[...] = q2


def _round_bf16_f32(x):
    u = jax.lax.bitcast_convert_type(x, jnp.uint32)
    r = (u + jnp.uint32(0x7FFF) + ((u >> 16) & jnp.uint32(1))) \
        & jnp.uint32(0xFFFF0000)
    return jax.lax.bitcast_convert_type(r, jnp.float32)


def _as_bf16(x):
    u = jax.lax.bitcast_convert_type(x, jnp.uint32)
    hi = (u >> 16).astype(jnp.uint16)
    return jax.lax.bitcast_convert_type(hi, jnp.bfloat16)


def _prep(pred, gt):
    q = pred.reshape(N, 3)
    g3 = gt.reshape(N, 3)
    bf16 = jnp.bfloat16

    qh = _round_bf16_f32(q)
    ql = _round_bf16_f32(q - qh)
    gh = _round_bf16_f32(g3)
    gl = _round_bf16_f32(g3 - gh)

    p = jnp.pad(q, ((0, 0), (0, KB - 3)))

    m2qh = _as_bf16(-2.0 * qh)
    m2ql = _as_bf16(-2.0 * ql)
    lb = jnp.zeros((N, KB), bf16)
    lb = lb.at[:, 0:3].set(m2qh)
    lb = lb.at[:, 3:6].set(1.0)

    lc = jnp.zeros((N, KC), bf16)
    lc = lc.at[:, 0:3].set(m2qh)
    lc = lc.at[:, 3:6].set(m2ql)
    lc = lc.at[:, 6:9].set(m2ql)

    r2 = jnp.sum(g3 * g3, axis=1)
    r2h = _round_bf16_f32(r2)
    r2l = _round_bf16_f32(r2 - r2h)
    r2l2 = _round_bf16_f32(r2 - r2h - r2l)
    rb = jnp.zeros((KB, N), bf16)
    rb = rb.at[0:3, :].set(_as_bf16(gh).T)
    rb = rb.at[3, :].set(_as_bf16(r2h))
    rb = rb.at[4, :].set(_as_bf16(r2l))
    rb = rb.at[5, :].set(_as_bf16(r2l2))

    glb = _as_bf16(gl)
    ghb = _as_bf16(gh)
    rc = jnp.zeros((KC, N), bf16)
    rc = rc.at[0:3, :].set(glb.T)
    rc = rc.at[3:6, :].set(ghb.T)
    rc = rc.at[6:9, :].set(glb.T)
    return lb, lc, rb, rc, p


@jax.jit
def _loss(pred, gt):
    lb, lc, rb, rc, p = _prep(pred, gt)
    tot = pl.pallas_call(
        _loss_kernel,
        grid=(N // TQ,),
        in_specs=[
            pl.BlockSpec((TQ, KB), lambda i: (i, 0)),
            pl.BlockSpec((TQ, KC), lambda i: (i, 0)),
            pl.BlockSpec((KB, N), lambda i: (0, 0)),
            pl.BlockSpec((KC, N), lambda i: (0, 0)),
            pl.BlockSpec((TQ, KB), lambda i: (i, 0)),
        ],
        out_specs=(pl.BlockSpec((TQ, 1), lambda i: (i, 0)),
                   pl.BlockSpec((TQ, 1), lambda i: (i, 0)),
                   pl.BlockSpec((TQ, 1), lambda i: (i, 0))),
        out_shape=(jax.ShapeDtypeStruct((N, 1), jnp.float32),
                   jax.ShapeDtypeStruct((N, 1), jnp.float32),
                   jax.ShapeDtypeStruct((N, 1), jnp.float32)),
    )(lb, lc, rb, rc, p)
    rowf, rowc, q2 = tot
    return (jnp.sum(rowf + rowc + q2) / (N * 3)).astype(jnp.float32)


def kernel(pred, gt):
    return _loss(pred, gt)


# speedup vs baseline: 4.7267x; 3.3372x over previous
import functools

import jax
import jax.numpy as jnp
from jax.experimental import pallas as pl

N = 8192
TQ = 256
NRC = 512
KB = 8
LW = 128


def _knn_kernel(lb_ref, rb_ref, out_ref):
    lb = lb_ref[...]

    lane = jax.lax.broadcasted_iota(jnp.int32, (TQ, LW), 1).astype(jnp.float32)

    def combine(a, b):
        af, ai = a
        bf, bi = b
        take = bf < af
        return jnp.where(take, bf, af), jnp.where(take, bi, ai)

    mf = mi = None
    for c in range(N // NRC):
        c0 = c * NRC
        kb = jax.lax.dot_general(
            lb, rb_ref[:, pl.ds(c0, NRC)], (((1,), (0,)), ((), ())),
            preferred_element_type=jnp.float32)
        gs = [(kb[:, g * LW:(g + 1) * LW], lane + float(c0 + g * LW))
              for g in range(NRC // LW)]
        t01 = combine(gs[0], gs[1])
        t23 = combine(gs[2], gs[3])
        t = combine(t01, t23)
        if mf is None:
            mf, mi = t
        else:
            mf, mi = combine((mf, mi), t)

    rowf = jnp.min(mf, axis=1, keepdims=True)
    rowi = jnp.min(jnp.where(mf == rowf, mi, jnp.inf),
                   axis=1, keepdims=True)
    out_ref[...] = rowi


def _round_bf16_f32(x):
    u = jax.lax.bitcast_convert_type(x, jnp.uint32)
    r = (u + jnp.uint32(0x7FFF) + ((u >> 16) & jnp.uint32(1))) \
        & jnp.uint32(0xFFFF0000)
    return jax.lax.bitcast_convert_type(r, jnp.float32)


def _as_bf16(x):
    u = jax.lax.bitcast_convert_type(x, jnp.uint32)
    hi = (u >> 16).astype(jnp.uint16)
    return jax.lax.bitcast_convert_type(hi, jnp.bfloat16)


def _prep(pred, gt):
    q = pred.reshape(N, 3)
    g3 = gt.reshape(N, 3)
    bf16 = jnp.bfloat16

    qh = _round_bf16_f32(q)
    gh = _round_bf16_f32(g3)

    m2qh = _as_bf16(-2.0 * qh)
    lb = jnp.zeros((N, KB), bf16)
    lb = lb.at[:, 0:3].set(m2qh)
    lb = lb.at[:, 3:6].set(1.0)

    r2 = jnp.sum(g3 * g3, axis=1)
    r2h = _round_bf16_f32(r2)
    r2l = _round_bf16_f32(r2 - r2h)
    r2l2 = _round_bf16_f32(r2 - r2h - r2l)
    rb = jnp.zeros((KB, N), bf16)
    rb = rb.at[0:3, :].set(_as_bf16(gh).T)
    rb = rb.at[3, :].set(_as_bf16(r2h))
    rb = rb.at[4, :].set(_as_bf16(r2l))
    rb = rb.at[5, :].set(_as_bf16(r2l2))
    return lb, rb


@jax.jit
def _loss(pred, gt):
    lb, rb = _prep(pred, gt)
    rowi = pl.pallas_call(
        _knn_kernel,
        grid=(N // TQ,),
        in_specs=[
            pl.BlockSpec((TQ, KB), lambda i: (i, 0)),
            pl.BlockSpec((KB, N), lambda i: (0, 0)),
        ],
        out_specs=pl.BlockSpec((TQ, 1), lambda i: (i, 0)),
        out_shape=jax.ShapeDtypeStruct((N, 1), jnp.float32),
    )(lb, rb)
    idx = rowi[:, 0].astype(jnp.int32).reshape(1, N)
    align_gt = jnp.take_along_axis(gt, idx[:, :, None], axis=1)
    return jnp.mean((pred - align_gt) ** 2)


def kernel(pred, gt):
    return _loss(pred, gt)
